# baseline (speedup 1.0000x reference)
import jax
import jax.numpy as jnp
from jax import lax
from jax.experimental import pallas as pl
from jax.experimental.pallas import tpu as pltpu
from jax.experimental.pallas import tpu_sc as plsc

VOCAB = 100000
EMBED = 64
CTX = 20
H = 256
BATCH = 1024
NIDX = BATCH * CTX

VT = 2048
NV = (VOCAB + VT - 1) // VT
LANES = 128
NCHUNK = VT // LANES

SC_CORES = 2
SC_SUBCORES = 16
NW = SC_CORES * SC_SUBCORES
B_PER_W = NIDX // NW


def _gather_body(table_hbm, idx_hbm, out_hbm, idx_v, rows_v, sem):
    wid = lax.axis_index("s") * SC_CORES + lax.axis_index("c")
    base = wid * B_PER_W
    pltpu.sync_copy(idx_hbm.at[pl.ds(base, B_PER_W)], idx_v)
    pltpu.async_copy(table_hbm.at[idx_v], rows_v, sem).wait()
    pltpu.sync_copy(rows_v, out_hbm.at[pl.ds(base, B_PER_W)])


def _sc_gather(emb_p, idx):
    mesh = plsc.VectorSubcoreMesh(core_axis_name="c", subcore_axis_name="s")
    k = pl.kernel(
        _gather_body,
        out_type=jax.ShapeDtypeStruct((NIDX, 2 * EMBED), jnp.float32),
        mesh=mesh,
        scratch_types=[
            pltpu.VMEM((B_PER_W,), jnp.int32),
            pltpu.VMEM((B_PER_W, 2 * EMBED), jnp.float32),
            pltpu.SemaphoreType.DMA,
        ],
    )
    return k(emb_p, idx)


def _h_body(e_ref, w1_ref, b1_ref, h_ref):
    e = e_ref[...].astype(jnp.bfloat16)
    w1 = w1_ref[...].astype(jnp.bfloat16)
    acc = lax.dot_general(e, w1, (((1,), (1,)), ((), ())),
                          preferred_element_type=jnp.float32)
    h_ref[...] = jnp.tanh(acc + b1_ref[...]).astype(jnp.bfloat16)


def _mlp_hidden(embeds, W1, b1):
    return pl.pallas_call(
        _h_body,
        out_shape=jax.ShapeDtypeStruct((BATCH, H), jnp.bfloat16),
    )(embeds, W1, b1.reshape(1, H))


def _p1_body(h_ref, w2_ref, m_ref, s_ref):
    j = pl.program_id(0)

    @pl.when(j == 0)
    def _():
        m_ref[...] = jnp.full((BATCH, LANES), -jnp.inf, jnp.float32)
        s_ref[...] = jnp.zeros((BATCH, LANES), jnp.float32)

    logits = lax.dot_general(h_ref[...], w2_ref[...], (((1,), (1,)), ((), ())),
                             preferred_element_type=jnp.float32)
    col = lax.broadcasted_iota(jnp.int32, (BATCH, VT), 1)
    logits = jnp.where(j * VT + col < VOCAB, logits, -jnp.inf)
    lg = logits.reshape(BATCH, NCHUNK, LANES)
    mc = jnp.max(lg, axis=1)
    m_prev = m_ref[...]
    s_prev = s_ref[...]
    m_new = jnp.maximum(m_prev, mc)
    e = jnp.exp(lg - m_new[:, None, :])
    m_ref[...] = m_new
    s_ref[...] = s_prev * jnp.exp(m_prev - m_new) + jnp.sum(e, axis=1)


def _lse_partials(h, w2b):
    return pl.pallas_call(
        _p1_body,
        grid=(NV,),
        in_specs=[
            pl.BlockSpec((BATCH, H), lambda j: (0, 0)),
            pl.BlockSpec((VT, H), lambda j: (j, 0)),
        ],
        out_specs=[
            pl.BlockSpec((BATCH, LANES), lambda j: (0, 0)),
            pl.BlockSpec((BATCH, LANES), lambda j: (0, 0)),
        ],
        out_shape=[
            jax.ShapeDtypeStruct((BATCH, LANES), jnp.float32),
            jax.ShapeDtypeStruct((BATCH, LANES), jnp.float32),
        ],
    )(h, w2b)


def _combine_body(m_ref, s_ref, c_ref):
    m = m_ref[...]
    s = s_ref[...]
    mrow = jnp.max(m, axis=1, keepdims=True)
    srow = jnp.sum(s * jnp.exp(m - mrow), axis=1, keepdims=True)
    c_ref[...] = mrow + jnp.log(srow)


def _combine(pm, ps):
    return pl.pallas_call(
        _combine_body,
        out_shape=jax.ShapeDtypeStruct((BATCH, 1), jnp.float32),
    )(pm, ps)


def _p2_body(h_ref, w2_ref, c_ref, o_ref):
    logits = lax.dot_general(h_ref[...], w2_ref[...], (((1,), (1,)), ((), ())),
                             preferred_element_type=jnp.float32)
    o_ref[...] = logits - c_ref[...]


def _project(h, w2b, c):
    return pl.pallas_call(
        _p2_body,
        grid=(NV,),
        in_specs=[
            pl.BlockSpec((BATCH, H), lambda j: (0, 0)),
            pl.BlockSpec((VT, H), lambda j: (j, 0)),
            pl.BlockSpec((BATCH, 1), lambda j: (0, 0)),
        ],
        out_specs=pl.BlockSpec((BATCH, VT), lambda j: (0, j)),
        out_shape=jax.ShapeDtypeStruct((BATCH, VOCAB), jnp.float32),
    )(h, w2b, c)


def kernel(inputs, emb, W1, b1, W2):
    idx = inputs.reshape(-1).astype(jnp.int32)
    emb_p = jnp.pad(emb, ((0, 0), (0, EMBED)))
    rows = _sc_gather(emb_p, idx)
    embeds = rows[:, :EMBED].reshape(BATCH, CTX * EMBED)
    w2b = W2.astype(jnp.bfloat16)
    h = _mlp_hidden(embeds, W1, b1)
    pm, ps = _lse_partials(h, w2b)
    c = _combine(pm, ps)
    return _project(h, w2b, c)
